# baseline (speedup 1.0000x reference)
import functools

import jax
import jax.numpy as jnp
from jax import lax
from jax.experimental import pallas as pl
from jax.experimental.pallas import tpu as pltpu
from jax.experimental.pallas import tpu_sc as plsc

N = 20000
MAX_KEEP = 300
IOU_THR = 0.7
NEG = float("-inf")
NW = 16
NPAD = 20480
SHARD = NPAD // NW
CHUNKS = SHARD // 16
BIG = 2**30

MB_CTR = 0
MB_CAND = 1
MB_BCAST = MB_CAND + 6 * NW
MB_STAMP = MB_BCAST + 6
MB_SIZE = MB_STAMP + 1


def _sc_nms(x1h, y1h, x2h, y2h, sch, keep_h, bx_h,
            vx1, vy1, vx2, vy2, vsc, va, keep_v, bx_v, mb):
    wid = lax.axis_index("s")
    base = wid * SHARD

    @pl.when(wid == 0)
    def _():
        for j in range(MB_SIZE):
            mb[j] = 0

    base8 = base
    pltpu.sync_copy(x1h.at[pl.ds(base8, SHARD)], vx1)
    pltpu.sync_copy(y1h.at[pl.ds(base8, SHARD)], vy1)
    pltpu.sync_copy(x2h.at[pl.ds(base8, SHARD)], vx2)
    pltpu.sync_copy(y2h.at[pl.ds(base8, SHARD)], vy2)
    pltpu.sync_copy(sch.at[pl.ds(base8, SHARD)], vsc)

    lanes = lax.broadcasted_iota(jnp.int32, (16,), 0)

    def area_chunk(c, _):
        off = c * 16
        va[pl.ds(off, 16)] = ((vx2[pl.ds(off, 16)] - vx1[pl.ds(off, 16)])
                              * (vy2[pl.ds(off, 16)] - vy1[pl.ds(off, 16)]))
        return 0

    lax.fori_loop(0, CHUNKS, area_chunk, 0)

    def rendezvous(target):
        plsc.fetch_and_add(mb.at[MB_CTR], 1, subcore_id=0)
        lax.while_loop(
            lambda v: v < target,
            lambda v: plsc.fetch_and_add(mb.at[MB_CTR], 0, subcore_id=0),
            jnp.int32(0))

    rendezvous(NW)

    def iter_body(i, carry):
        wx1, wy1, wx2, wy2, wiv = carry
        warea = (wx2 - wx1) * (wy2 - wy1)

        own = (wiv >= base) & (wiv < base + SHARD) & (lanes == 0)
        plsc.store_scatter(vsc, [jnp.clip(wiv - base, 0, SHARD - 1)],
                           jnp.full((16,), NEG, jnp.float32), mask=own)

        def chunk(c, acc):
            bs, bi, gi = acc
            off = c * 16
            cx1 = vx1[pl.ds(off, 16)]
            cy1 = vy1[pl.ds(off, 16)]
            cx2 = vx2[pl.ds(off, 16)]
            cy2 = vy2[pl.ds(off, 16)]
            s = vsc[pl.ds(off, 16)]
            a = va[pl.ds(off, 16)]
            ix1 = jnp.maximum(wx1, cx1)
            iy1 = jnp.maximum(wy1, cy1)
            ix2 = jnp.minimum(wx2, cx2)
            iy2 = jnp.minimum(wy2, cy2)
            iw = jnp.maximum(ix2 - ix1, 0.0)
            ih = jnp.maximum(iy2 - iy1, 0.0)
            inter = iw * ih
            union = warea + a - inter
            supp = inter > IOU_THR * union
            s = jnp.where(supp, NEG, s)
            vsc[pl.ds(off, 16)] = s
            better = s > bs
            bs = jnp.maximum(bs, s)
            bi = jnp.where(better, gi, bi)
            return bs, bi, gi + 16

        bs0 = jnp.full((16,), NEG, jnp.float32)
        bi0 = jnp.full((16,), BIG, jnp.int32)
        bs, bi, _ = lax.fori_loop(0, CHUNKS, chunk, (bs0, bi0, base + lanes))

        m = jnp.max(bs)
        li = jnp.min(jnp.where(bs == m, bi, BIG))
        p = jnp.clip(li - base, 0, SHARD - 1)
        pf = jnp.full((16,), p, jnp.int32)
        c1 = plsc.bitcast(plsc.load_gather(vx1, [pf]), jnp.int32)
        c2 = plsc.bitcast(plsc.load_gather(vy1, [pf]), jnp.int32)
        c3 = plsc.bitcast(plsc.load_gather(vx2, [pf]), jnp.int32)
        c4 = plsc.bitcast(plsc.load_gather(vy2, [pf]), jnp.int32)
        key = jnp.where(m > NEG,
                        jnp.max(plsc.bitcast(jnp.full((16,), m, jnp.float32),
                                             jnp.int32)),
                        jnp.int32(-1))
        payload = (key, li, jnp.max(c1), jnp.max(c2), jnp.max(c3), jnp.max(c4))

        slot0 = MB_CAND + wid * 6
        for k in range(6):
            plsc.fetch_and_add(mb.at[slot0 + k], payload[k], subcore_id=0)

        rendezvous(NW * (i + 2))

        @pl.when(wid == 0)
        def _():
            wkey = mb[MB_CAND + 0]
            widx = mb[MB_CAND + 1]
            w1 = mb[MB_CAND + 2]
            w2 = mb[MB_CAND + 3]
            w3 = mb[MB_CAND + 4]
            w4 = mb[MB_CAND + 5]
            for t in range(1, NW):
                tk = mb[MB_CAND + t * 6]
                ti = mb[MB_CAND + t * 6 + 1]
                better = (tk > wkey) | ((tk == wkey) & (ti < widx))
                wkey = jnp.where(better, tk, wkey)
                widx = jnp.where(better, ti, widx)
                w1 = jnp.where(better, mb[MB_CAND + t * 6 + 2], w1)
                w2 = jnp.where(better, mb[MB_CAND + t * 6 + 3], w2)
                w3 = jnp.where(better, mb[MB_CAND + t * 6 + 4], w3)
                w4 = jnp.where(better, mb[MB_CAND + t * 6 + 5], w4)
            for j in range(6 * NW):
                mb[MB_CAND + j] = 0
            mb[MB_BCAST + 0] = wkey
            mb[MB_BCAST + 1] = widx
            mb[MB_BCAST + 2] = w1
            mb[MB_BCAST + 3] = w2
            mb[MB_BCAST + 4] = w3
            mb[MB_BCAST + 5] = w4
            mb[MB_STAMP] = i + 1

        def poll_cond(st):
            return st[0] != i + 1

        def poll_body(st):
            stv = plsc.fetch_and_add(mb.at[MB_STAMP], 0, subcore_id=0)
            b0 = plsc.fetch_and_add(mb.at[MB_BCAST + 0], 0, subcore_id=0)
            b1 = plsc.fetch_and_add(mb.at[MB_BCAST + 1], 0, subcore_id=0)
            b2 = plsc.fetch_and_add(mb.at[MB_BCAST + 2], 0, subcore_id=0)
            b3 = plsc.fetch_and_add(mb.at[MB_BCAST + 3], 0, subcore_id=0)
            b4 = plsc.fetch_and_add(mb.at[MB_BCAST + 4], 0, subcore_id=0)
            b5 = plsc.fetch_and_add(mb.at[MB_BCAST + 5], 0, subcore_id=0)
            return (stv, b0, b1, b2, b3, b4, b5)

        z = jnp.int32(0)
        st = lax.while_loop(poll_cond, poll_body, (z, z, z, z, z, z, z))
        wkey, widx = st[1], st[2]
        anyv = wkey >= 0
        nx1 = jnp.where(anyv,
                        plsc.bitcast(jnp.full((16,), st[3], jnp.int32),
                                     jnp.float32), 0.0)
        ny1 = jnp.where(anyv,
                        plsc.bitcast(jnp.full((16,), st[4], jnp.int32),
                                     jnp.float32), 0.0)
        nx2 = jnp.where(anyv,
                        plsc.bitcast(jnp.full((16,), st[5], jnp.int32),
                                     jnp.float32), 0.0)
        ny2 = jnp.where(anyv,
                        plsc.bitcast(jnp.full((16,), st[6], jnp.int32),
                                     jnp.float32), 0.0)
        wiv_n = jnp.where(anyv, jnp.full((16,), widx, jnp.int32),
                          jnp.full((16,), -1, jnp.int32))

        @pl.when(wid == 0)
        def _():
            kv = jnp.where(anyv, widx, jnp.int32(-1))
            plsc.store_scatter(keep_v, [jnp.full((16,), i, jnp.int32)],
                               jnp.full((16,), kv, jnp.int32),
                               mask=lanes == 0)
            coords = jnp.where(lanes == 0, nx1,
                     jnp.where(lanes == 1, ny1,
                     jnp.where(lanes == 2, nx2, ny2)))
            plsc.store_scatter(bx_v, [jnp.full((16,), i, jnp.int32), lanes],
                               coords, mask=lanes < 4)

        return (nx1, ny1, nx2, ny2, wiv_n)

    init = (jnp.zeros((16,), jnp.float32),
            jnp.zeros((16,), jnp.float32),
            jnp.zeros((16,), jnp.float32),
            jnp.zeros((16,), jnp.float32),
            jnp.full((16,), -1, jnp.int32))
    lax.fori_loop(0, MAX_KEEP, iter_body, init)

    @pl.when(wid == 0)
    def _():
        pltpu.sync_copy(keep_v, keep_h)
        pltpu.sync_copy(bx_v, bx_h)


@jax.jit
def kernel(boxes, scores):
    pad = NPAD - N
    x1 = jnp.pad(boxes[:, 0], (0, pad))
    y1 = jnp.pad(boxes[:, 1], (0, pad))
    x2 = jnp.pad(boxes[:, 2], (0, pad))
    y2 = jnp.pad(boxes[:, 3], (0, pad))
    sc = jnp.pad(scores, (0, pad), constant_values=NEG)

    mesh = plsc.VectorSubcoreMesh(core_axis_name="c", subcore_axis_name="s",
                                  num_cores=1, num_subcores=NW)
    f = pl.kernel(
        _sc_nms,
        out_type=[
            jax.ShapeDtypeStruct((MAX_KEEP,), jnp.int32),
            jax.ShapeDtypeStruct((MAX_KEEP, 4), jnp.float32),
        ],
        mesh=mesh,
        compiler_params=pltpu.CompilerParams(needs_layout_passes=False),
        scratch_types=[
            pltpu.VMEM((SHARD,), jnp.float32),
            pltpu.VMEM((SHARD,), jnp.float32),
            pltpu.VMEM((SHARD,), jnp.float32),
            pltpu.VMEM((SHARD,), jnp.float32),
            pltpu.VMEM((SHARD,), jnp.float32),
            pltpu.VMEM((SHARD,), jnp.float32),
            pltpu.VMEM((MAX_KEEP,), jnp.int32),
            pltpu.VMEM((MAX_KEEP, 4), jnp.float32),
            pltpu.SMEM((MB_SIZE,), jnp.int32),
        ],
    )
    keep, kept_boxes = f(x1, y1, x2, y2, sc)
    return kept_boxes, keep


# speedup vs baseline: 1.1709x; 1.1709x over previous
import functools

import jax
import jax.numpy as jnp
from jax import lax
from jax.experimental import pallas as pl
from jax.experimental.pallas import tpu as pltpu
from jax.experimental.pallas import tpu_sc as plsc

N = 20000
MAX_KEEP = 300
IOU_THR = 0.7
NEG = float("-inf")
NW = 16
NPAD = 20480
SHARD = NPAD // NW
CHUNKS = SHARD // 16
BIG = 2**30

MB_CTR = 0
MB_CAND = 1
MB_BCAST = MB_CAND + 6 * NW
MB_STAMP = MB_BCAST + 6
MB_SIZE = MB_STAMP + 1


def _sc_nms(x1h, y1h, x2h, y2h, sch, keep_h, bx_h,
            vx1, vy1, vx2, vy2, vsc, va, keep_v, bx_v, mb):
    wid = lax.axis_index("s")
    base = wid * SHARD

    @pl.when(wid == 0)
    def _():
        for j in range(MB_SIZE):
            mb[j] = 0

    base8 = base
    pltpu.sync_copy(x1h.at[pl.ds(base8, SHARD)], vx1)
    pltpu.sync_copy(y1h.at[pl.ds(base8, SHARD)], vy1)
    pltpu.sync_copy(x2h.at[pl.ds(base8, SHARD)], vx2)
    pltpu.sync_copy(y2h.at[pl.ds(base8, SHARD)], vy2)
    pltpu.sync_copy(sch.at[pl.ds(base8, SHARD)], vsc)

    lanes = lax.broadcasted_iota(jnp.int32, (16,), 0)

    def area_chunk(c, _):
        off = c * 16
        va[pl.ds(off, 16)] = ((vx2[pl.ds(off, 16)] - vx1[pl.ds(off, 16)])
                              * (vy2[pl.ds(off, 16)] - vy1[pl.ds(off, 16)]))
        return 0

    lax.fori_loop(0, CHUNKS, area_chunk, 0)

    def rendezvous(target):
        plsc.fetch_and_add(mb.at[MB_CTR], 1, subcore_id=0)
        lax.while_loop(
            lambda v: v < target,
            lambda v: plsc.fetch_and_add(mb.at[MB_CTR], 0, subcore_id=0),
            jnp.int32(0))

    rendezvous(NW)

    def iter_body(i, carry):
        wx1, wy1, wx2, wy2, wiv = carry
        warea = (wx2 - wx1) * (wy2 - wy1)

        own = (wiv >= base) & (wiv < base + SHARD) & (lanes == 0)
        plsc.store_scatter(vsc, [jnp.clip(wiv - base, 0, SHARD - 1)],
                           jnp.full((16,), NEG, jnp.float32), mask=own)

        def one(off, gi, bs, bi):
            cx1 = vx1[pl.ds(off, 16)]
            cy1 = vy1[pl.ds(off, 16)]
            cx2 = vx2[pl.ds(off, 16)]
            cy2 = vy2[pl.ds(off, 16)]
            s = vsc[pl.ds(off, 16)]
            a = va[pl.ds(off, 16)]
            ix1 = jnp.maximum(wx1, cx1)
            iy1 = jnp.maximum(wy1, cy1)
            ix2 = jnp.minimum(wx2, cx2)
            iy2 = jnp.minimum(wy2, cy2)
            iw = jnp.maximum(ix2 - ix1, 0.0)
            ih = jnp.maximum(iy2 - iy1, 0.0)
            inter = iw * ih
            union = warea + a - inter
            supp = inter / union > IOU_THR
            s = jnp.where(supp, NEG, s)
            vsc[pl.ds(off, 16)] = s
            better = s > bs
            bs = jnp.maximum(bs, s)
            bi = jnp.where(better, gi, bi)
            return bs, bi

        def chunk(c, acc):
            bs, bi, gi = acc
            off = c * 32
            bs, bi = one(off, gi, bs, bi)
            bs, bi = one(off + 16, gi + 16, bs, bi)
            return bs, bi, gi + 32

        bs0 = jnp.full((16,), NEG, jnp.float32)
        bi0 = jnp.full((16,), BIG, jnp.int32)
        bs, bi, _ = lax.fori_loop(0, CHUNKS // 2, chunk,
                                  (bs0, bi0, base + lanes))

        m = jnp.max(bs)
        li = jnp.min(jnp.where(bs == m, bi, BIG))
        p = jnp.clip(li - base, 0, SHARD - 1)
        pf = jnp.full((16,), p, jnp.int32)
        c1 = plsc.bitcast(plsc.load_gather(vx1, [pf]), jnp.int32)
        c2 = plsc.bitcast(plsc.load_gather(vy1, [pf]), jnp.int32)
        c3 = plsc.bitcast(plsc.load_gather(vx2, [pf]), jnp.int32)
        c4 = plsc.bitcast(plsc.load_gather(vy2, [pf]), jnp.int32)
        key = jnp.where(m > NEG,
                        jnp.max(plsc.bitcast(jnp.full((16,), m, jnp.float32),
                                             jnp.int32)),
                        jnp.int32(-1))
        payload = (key, li, jnp.max(c1), jnp.max(c2), jnp.max(c3), jnp.max(c4))

        slot0 = MB_CAND + wid * 6
        for k in range(6):
            plsc.fetch_and_add(mb.at[slot0 + k], payload[k], subcore_id=0)

        rendezvous(NW * (i + 2))

        @pl.when(wid == 0)
        def _():
            wkey = mb[MB_CAND + 0]
            widx = mb[MB_CAND + 1]
            w1 = mb[MB_CAND + 2]
            w2 = mb[MB_CAND + 3]
            w3 = mb[MB_CAND + 4]
            w4 = mb[MB_CAND + 5]
            for t in range(1, NW):
                tk = mb[MB_CAND + t * 6]
                ti = mb[MB_CAND + t * 6 + 1]
                better = (tk > wkey) | ((tk == wkey) & (ti < widx))
                wkey = jnp.where(better, tk, wkey)
                widx = jnp.where(better, ti, widx)
                w1 = jnp.where(better, mb[MB_CAND + t * 6 + 2], w1)
                w2 = jnp.where(better, mb[MB_CAND + t * 6 + 3], w2)
                w3 = jnp.where(better, mb[MB_CAND + t * 6 + 4], w3)
                w4 = jnp.where(better, mb[MB_CAND + t * 6 + 5], w4)
            for j in range(6 * NW):
                mb[MB_CAND + j] = 0
            mb[MB_BCAST + 0] = wkey
            mb[MB_BCAST + 1] = widx
            mb[MB_BCAST + 2] = w1
            mb[MB_BCAST + 3] = w2
            mb[MB_BCAST + 4] = w3
            mb[MB_BCAST + 5] = w4
            mb[MB_STAMP] = i + 1

        lax.while_loop(
            lambda v: v != i + 1,
            lambda v: plsc.fetch_and_add(mb.at[MB_STAMP], 0, subcore_id=0),
            jnp.int32(0))
        st = (jnp.int32(0),
              plsc.fetch_and_add(mb.at[MB_BCAST + 0], 0, subcore_id=0),
              plsc.fetch_and_add(mb.at[MB_BCAST + 1], 0, subcore_id=0),
              plsc.fetch_and_add(mb.at[MB_BCAST + 2], 0, subcore_id=0),
              plsc.fetch_and_add(mb.at[MB_BCAST + 3], 0, subcore_id=0),
              plsc.fetch_and_add(mb.at[MB_BCAST + 4], 0, subcore_id=0),
              plsc.fetch_and_add(mb.at[MB_BCAST + 5], 0, subcore_id=0))
        wkey, widx = st[1], st[2]
        anyv = wkey >= 0
        nx1 = jnp.where(anyv,
                        plsc.bitcast(jnp.full((16,), st[3], jnp.int32),
                                     jnp.float32), 0.0)
        ny1 = jnp.where(anyv,
                        plsc.bitcast(jnp.full((16,), st[4], jnp.int32),
                                     jnp.float32), 0.0)
        nx2 = jnp.where(anyv,
                        plsc.bitcast(jnp.full((16,), st[5], jnp.int32),
                                     jnp.float32), 0.0)
        ny2 = jnp.where(anyv,
                        plsc.bitcast(jnp.full((16,), st[6], jnp.int32),
                                     jnp.float32), 0.0)
        wiv_n = jnp.where(anyv, jnp.full((16,), widx, jnp.int32),
                          jnp.full((16,), -1, jnp.int32))

        @pl.when(wid == 0)
        def _():
            kv = jnp.where(anyv, widx, jnp.int32(-1))
            plsc.store_scatter(keep_v, [jnp.full((16,), i, jnp.int32)],
                               jnp.full((16,), kv, jnp.int32),
                               mask=lanes == 0)
            coords = jnp.where(lanes == 0, nx1,
                     jnp.where(lanes == 1, ny1,
                     jnp.where(lanes == 2, nx2, ny2)))
            plsc.store_scatter(bx_v, [jnp.full((16,), i, jnp.int32), lanes],
                               coords, mask=lanes < 4)

        return (nx1, ny1, nx2, ny2, wiv_n)

    init = (jnp.zeros((16,), jnp.float32),
            jnp.zeros((16,), jnp.float32),
            jnp.zeros((16,), jnp.float32),
            jnp.zeros((16,), jnp.float32),
            jnp.full((16,), -1, jnp.int32))
    lax.fori_loop(0, MAX_KEEP, iter_body, init)

    @pl.when(wid == 0)
    def _():
        pltpu.sync_copy(keep_v, keep_h)
        pltpu.sync_copy(bx_v, bx_h)


@jax.jit
def kernel(boxes, scores):
    pad = NPAD - N
    x1 = jnp.pad(boxes[:, 0], (0, pad))
    y1 = jnp.pad(boxes[:, 1], (0, pad))
    x2 = jnp.pad(boxes[:, 2], (0, pad))
    y2 = jnp.pad(boxes[:, 3], (0, pad))
    sc = jnp.pad(scores, (0, pad), constant_values=NEG)

    mesh = plsc.VectorSubcoreMesh(core_axis_name="c", subcore_axis_name="s",
                                  num_cores=1, num_subcores=NW)
    f = pl.kernel(
        _sc_nms,
        out_type=[
            jax.ShapeDtypeStruct((MAX_KEEP,), jnp.int32),
            jax.ShapeDtypeStruct((MAX_KEEP, 4), jnp.float32),
        ],
        mesh=mesh,
        compiler_params=pltpu.CompilerParams(needs_layout_passes=False),
        scratch_types=[
            pltpu.VMEM((SHARD,), jnp.float32),
            pltpu.VMEM((SHARD,), jnp.float32),
            pltpu.VMEM((SHARD,), jnp.float32),
            pltpu.VMEM((SHARD,), jnp.float32),
            pltpu.VMEM((SHARD,), jnp.float32),
            pltpu.VMEM((SHARD,), jnp.float32),
            pltpu.VMEM((MAX_KEEP,), jnp.int32),
            pltpu.VMEM((MAX_KEEP, 4), jnp.float32),
            pltpu.SMEM((MB_SIZE,), jnp.int32),
        ],
    )
    keep, kept_boxes = f(x1, y1, x2, y2, sc)
    return kept_boxes, keep


# speedup vs baseline: 1.2877x; 1.0997x over previous
import functools

import jax
import jax.numpy as jnp
from jax import lax
from jax.experimental import pallas as pl
from jax.experimental.pallas import tpu as pltpu
from jax.experimental.pallas import tpu_sc as plsc

N = 20000
MAX_KEEP = 300
IOU_THR = 0.7
NEG = float("-inf")
NW = 16
NPAD = 20480
SHARD = NPAD // NW
CHUNKS = SHARD // 16
BIG = 2**30

MB_CTR = 0
MB_CAND = 1
MB_BCAST = MB_CAND + 6 * NW
MB_STAMP = MB_BCAST + 6
MB_SIZE = MB_STAMP + 1


def _sc_nms(x1h, y1h, x2h, y2h, sch, keep_h, bx_h,
            vx1, vy1, vx2, vy2, vsc, va, keep_v, bx_v, mb):
    wid = lax.axis_index("s")
    base = wid * SHARD

    @pl.when(wid == 0)
    def _():
        for j in range(MB_SIZE):
            mb[j] = 0

    base8 = base
    pltpu.sync_copy(x1h.at[pl.ds(base8, SHARD)], vx1)
    pltpu.sync_copy(y1h.at[pl.ds(base8, SHARD)], vy1)
    pltpu.sync_copy(x2h.at[pl.ds(base8, SHARD)], vx2)
    pltpu.sync_copy(y2h.at[pl.ds(base8, SHARD)], vy2)
    pltpu.sync_copy(sch.at[pl.ds(base8, SHARD)], vsc)

    lanes = lax.broadcasted_iota(jnp.int32, (16,), 0)

    def area_chunk(c, _):
        off = c * 16
        va[pl.ds(off, 16)] = ((vx2[pl.ds(off, 16)] - vx1[pl.ds(off, 16)])
                              * (vy2[pl.ds(off, 16)] - vy1[pl.ds(off, 16)]))
        return 0

    lax.fori_loop(0, CHUNKS, area_chunk, 0)

    def arrive(target, spin):
        plsc.fetch_and_add(mb.at[MB_CTR], 1, subcore_id=0)

        @pl.when(spin)
        def _():
            lax.while_loop(
                lambda v: v < target,
                lambda v: plsc.fetch_and_add(mb.at[MB_CTR], 0, subcore_id=0),
                jnp.int32(0))

    arrive(NW, True)

    def iter_body(i, carry):
        wx1, wy1, wx2, wy2, wiv, prev = carry
        warea = (wx2 - wx1) * (wy2 - wy1)

        own = (wiv >= base) & (wiv < base + SHARD) & (lanes == 0)
        plsc.store_scatter(vsc, [jnp.clip(wiv - base, 0, SHARD - 1)],
                           jnp.full((16,), NEG, jnp.float32), mask=own)

        def one(off, gi, bs, bi):
            cx1 = vx1[pl.ds(off, 16)]
            cy1 = vy1[pl.ds(off, 16)]
            cx2 = vx2[pl.ds(off, 16)]
            cy2 = vy2[pl.ds(off, 16)]
            s = vsc[pl.ds(off, 16)]
            a = va[pl.ds(off, 16)]
            ix1 = jnp.maximum(wx1, cx1)
            iy1 = jnp.maximum(wy1, cy1)
            ix2 = jnp.minimum(wx2, cx2)
            iy2 = jnp.minimum(wy2, cy2)
            iw = jnp.maximum(ix2 - ix1, 0.0)
            ih = jnp.maximum(iy2 - iy1, 0.0)
            inter = iw * ih
            union = warea + a - inter
            supp = inter / union > IOU_THR
            s = jnp.where(supp, NEG, s)
            vsc[pl.ds(off, 16)] = s
            better = s > bs
            bs = jnp.maximum(bs, s)
            bi = jnp.where(better, gi, bi)
            return bs, bi

        def chunk(c, acc):
            bs, bi, gi = acc
            off = c * 64
            bs, bi = one(off, gi, bs, bi)
            bs, bi = one(off + 16, gi + 16, bs, bi)
            bs, bi = one(off + 32, gi + 32, bs, bi)
            bs, bi = one(off + 48, gi + 48, bs, bi)
            return bs, bi, gi + 64

        bs0 = jnp.full((16,), NEG, jnp.float32)
        bi0 = jnp.full((16,), BIG, jnp.int32)
        bs, bi, _ = lax.fori_loop(0, CHUNKS // 4, chunk,
                                  (bs0, bi0, base + lanes))

        m = jnp.max(bs)
        li = jnp.min(jnp.where(bs == m, bi, BIG))
        p = jnp.clip(li - base, 0, SHARD - 1)
        pf = jnp.full((16,), p, jnp.int32)
        c1 = plsc.bitcast(plsc.load_gather(vx1, [pf]), jnp.int32)
        c2 = plsc.bitcast(plsc.load_gather(vy1, [pf]), jnp.int32)
        c3 = plsc.bitcast(plsc.load_gather(vx2, [pf]), jnp.int32)
        c4 = plsc.bitcast(plsc.load_gather(vy2, [pf]), jnp.int32)
        key = jnp.where(m > NEG,
                        jnp.max(plsc.bitcast(jnp.full((16,), m, jnp.float32),
                                             jnp.int32)),
                        jnp.int32(-1))
        payload = (key, li, jnp.max(c1), jnp.max(c2), jnp.max(c3), jnp.max(c4))

        slot0 = MB_CAND + wid * 6
        for k in range(6):
            plsc.fetch_and_add(mb.at[slot0 + k], payload[k] - prev[k],
                               subcore_id=0)

        arrive(NW * (i + 2), wid == 0)

        @pl.when(wid == 0)
        def _():
            wkey = mb[MB_CAND + 0]
            widx = mb[MB_CAND + 1]
            w1 = mb[MB_CAND + 2]
            w2 = mb[MB_CAND + 3]
            w3 = mb[MB_CAND + 4]
            w4 = mb[MB_CAND + 5]
            for t in range(1, NW):
                tk = mb[MB_CAND + t * 6]
                ti = mb[MB_CAND + t * 6 + 1]
                better = (tk > wkey) | ((tk == wkey) & (ti < widx))
                wkey = jnp.where(better, tk, wkey)
                widx = jnp.where(better, ti, widx)
                w1 = jnp.where(better, mb[MB_CAND + t * 6 + 2], w1)
                w2 = jnp.where(better, mb[MB_CAND + t * 6 + 3], w2)
                w3 = jnp.where(better, mb[MB_CAND + t * 6 + 4], w3)
                w4 = jnp.where(better, mb[MB_CAND + t * 6 + 5], w4)
            mb[MB_BCAST + 0] = wkey
            mb[MB_BCAST + 1] = widx
            mb[MB_BCAST + 2] = w1
            mb[MB_BCAST + 3] = w2
            mb[MB_BCAST + 4] = w3
            mb[MB_BCAST + 5] = w4
            mb[MB_STAMP] = i + 1

        lax.while_loop(
            lambda v: v != i + 1,
            lambda v: plsc.fetch_and_add(mb.at[MB_STAMP], 0, subcore_id=0),
            jnp.int32(0))
        st = (jnp.int32(0),
              plsc.fetch_and_add(mb.at[MB_BCAST + 0], 0, subcore_id=0),
              plsc.fetch_and_add(mb.at[MB_BCAST + 1], 0, subcore_id=0),
              plsc.fetch_and_add(mb.at[MB_BCAST + 2], 0, subcore_id=0),
              plsc.fetch_and_add(mb.at[MB_BCAST + 3], 0, subcore_id=0),
              plsc.fetch_and_add(mb.at[MB_BCAST + 4], 0, subcore_id=0),
              plsc.fetch_and_add(mb.at[MB_BCAST + 5], 0, subcore_id=0))
        wkey, widx = st[1], st[2]
        anyv = wkey >= 0
        nx1 = jnp.where(anyv,
                        plsc.bitcast(jnp.full((16,), st[3], jnp.int32),
                                     jnp.float32), 0.0)
        ny1 = jnp.where(anyv,
                        plsc.bitcast(jnp.full((16,), st[4], jnp.int32),
                                     jnp.float32), 0.0)
        nx2 = jnp.where(anyv,
                        plsc.bitcast(jnp.full((16,), st[5], jnp.int32),
                                     jnp.float32), 0.0)
        ny2 = jnp.where(anyv,
                        plsc.bitcast(jnp.full((16,), st[6], jnp.int32),
                                     jnp.float32), 0.0)
        wiv_n = jnp.where(anyv, jnp.full((16,), widx, jnp.int32),
                          jnp.full((16,), -1, jnp.int32))

        @pl.when(wid == 0)
        def _():
            kv = jnp.where(anyv, widx, jnp.int32(-1))
            plsc.store_scatter(keep_v, [jnp.full((16,), i, jnp.int32)],
                               jnp.full((16,), kv, jnp.int32),
                               mask=lanes == 0)
            coords = jnp.where(lanes == 0, nx1,
                     jnp.where(lanes == 1, ny1,
                     jnp.where(lanes == 2, nx2, ny2)))
            plsc.store_scatter(bx_v, [jnp.full((16,), i, jnp.int32), lanes],
                               coords, mask=lanes < 4)

        return (nx1, ny1, nx2, ny2, wiv_n, payload)

    z = jnp.int32(0)
    init = (jnp.zeros((16,), jnp.float32),
            jnp.zeros((16,), jnp.float32),
            jnp.zeros((16,), jnp.float32),
            jnp.zeros((16,), jnp.float32),
            jnp.full((16,), -1, jnp.int32),
            (z, z, z, z, z, z))
    lax.fori_loop(0, MAX_KEEP, iter_body, init)

    @pl.when(wid == 0)
    def _():
        pltpu.sync_copy(keep_v, keep_h)
        pltpu.sync_copy(bx_v, bx_h)


@jax.jit
def kernel(boxes, scores):
    pad = NPAD - N
    x1 = jnp.pad(boxes[:, 0], (0, pad))
    y1 = jnp.pad(boxes[:, 1], (0, pad))
    x2 = jnp.pad(boxes[:, 2], (0, pad))
    y2 = jnp.pad(boxes[:, 3], (0, pad))
    sc = jnp.pad(scores, (0, pad), constant_values=NEG)

    mesh = plsc.VectorSubcoreMesh(core_axis_name="c", subcore_axis_name="s",
                                  num_cores=1, num_subcores=NW)
    f = pl.kernel(
        _sc_nms,
        out_type=[
            jax.ShapeDtypeStruct((MAX_KEEP,), jnp.int32),
            jax.ShapeDtypeStruct((MAX_KEEP, 4), jnp.float32),
        ],
        mesh=mesh,
        compiler_params=pltpu.CompilerParams(needs_layout_passes=False),
        scratch_types=[
            pltpu.VMEM((SHARD,), jnp.float32),
            pltpu.VMEM((SHARD,), jnp.float32),
            pltpu.VMEM((SHARD,), jnp.float32),
            pltpu.VMEM((SHARD,), jnp.float32),
            pltpu.VMEM((SHARD,), jnp.float32),
            pltpu.VMEM((SHARD,), jnp.float32),
            pltpu.VMEM((MAX_KEEP,), jnp.int32),
            pltpu.VMEM((MAX_KEEP, 4), jnp.float32),
            pltpu.SMEM((MB_SIZE,), jnp.int32),
        ],
    )
    keep, kept_boxes = f(x1, y1, x2, y2, sc)
    return kept_boxes, keep
